# baseline (speedup 1.0000x reference)
import functools

import jax
import jax.numpy as jnp
from jax.experimental import pallas as pl
from jax.experimental.pallas import tpu as pltpu
from jax.experimental.pallas import tpu_sc as plsc

_NA = 64
_ND = 4096
_NQ = 1024
_NR = _ND + _NQ
_RB = 512
_CB = 512
_NW = 32
_RPW = _NR // _NW
_LANES = 16


def _logits_body(xt_ref, w_ref, b_ref, out_ref):
    lt = jax.lax.dot_general(
        w_ref[...], xt_ref[...], (((1,), (0,)), ((), ())),
        preferred_element_type=jnp.float32)
    out_ref[...] = lt + b_ref[...]


_CHUNK = 128
_NCHUNK = _NR // _CHUNK


def _tree_count(srcs, va):
    terms = [jnp.where(s > va, 1.0, 0.0) for s in srcs]
    while len(terms) > 1:
        nxt = [terms[i] + terms[i + 1] for i in range(0, len(terms) - 1, 2)]
        if len(terms) % 2:
            nxt.append(terms[-1])
        terms = nxt
    return terms[0]


def _sc_rank_body(lt_hbm, out_hbm, lt_v, rk_v):
    c = jax.lax.axis_index("c")
    s = jax.lax.axis_index("s")
    wid = s * 2 + c

    def do_chunk(chunk):
        base = chunk * _CHUNK
        pltpu.sync_copy(lt_hbm.at[:, pl.ds(base, _CHUNK)], lt_v)

        def group(g, carry):
            col = g * _LANES

            def half(h):
                srcs = [lt_v[h * 32 + j, pl.ds(col, _LANES)] for j in range(32)]

                def target(a, carry2):
                    va = lt_v[a, pl.ds(col, _LANES)]
                    cnt = _tree_count(srcs, va)
                    if h == 0:
                        rk_v[a, pl.ds(col, _LANES)] = cnt + 1.0
                    else:
                        rk_v[a, pl.ds(col, _LANES)] = (
                            rk_v[a, pl.ds(col, _LANES)] + cnt)
                    return carry2

                jax.lax.fori_loop(0, _NA, target, 0)

            half(0)
            half(1)
            return carry

        jax.lax.fori_loop(0, _CHUNK // _LANES, group, 0)
        pltpu.sync_copy(rk_v, out_hbm.at[:, pl.ds(base, _CHUNK)])

    do_chunk(wid)

    @pl.when(wid + _NW < _NCHUNK)
    def _():
        do_chunk(wid + _NW)


def _sc_ranks(lt):
    mesh = plsc.VectorSubcoreMesh(core_axis_name="c", subcore_axis_name="s")
    return pl.kernel(
        _sc_rank_body,
        out_type=jax.ShapeDtypeStruct((_NA, _NR), jnp.float32),
        mesh=mesh,
        scratch_types=[
            pltpu.VMEM((_NA, _CHUNK), jnp.float32),
            pltpu.VMEM((_NA, _CHUNK), jnp.float32),
        ],
    )(lt)


def _mm_body(q_ref, d_ref, out_ref):
    out_ref[...] = jax.lax.dot_general(
        q_ref[...], d_ref[...], (((1,), (0,)), ((), ())),
        preferred_element_type=jnp.float32)


def kernel(data, query, W, b):
    with jax.enable_x64(False):
        return _kernel32(data, query, W, b)


def _kernel32(data, query, W, b):
    rows_t = jnp.concatenate([data, query], axis=0).T
    logits_t = pl.pallas_call(
        _logits_body,
        grid=(_NR // _RB,),
        in_specs=[
            pl.BlockSpec((128, _RB), lambda i: (0, i)),
            pl.BlockSpec((_NA, 128), lambda i: (0, 0)),
            pl.BlockSpec((_NA, 1), lambda i: (0, 0)),
        ],
        out_specs=pl.BlockSpec((_NA, _RB), lambda i: (0, i)),
        out_shape=jax.ShapeDtypeStruct((_NA, _NR), jnp.float32),
    )(rows_t, W, b.reshape(_NA, 1))
    ranks_t = _sc_ranks(logits_t).astype(jnp.bfloat16)
    d_rank_t = ranks_t[:, :_ND]
    q_rank = ranks_t[:, _ND:].T
    out = pl.pallas_call(
        _mm_body,
        grid=(_ND // _CB,),
        in_specs=[
            pl.BlockSpec((_NQ, _NA), lambda j: (0, 0)),
            pl.BlockSpec((_NA, _CB), lambda j: (0, j)),
        ],
        out_specs=pl.BlockSpec((_NQ, _CB), lambda j: (0, j)),
        out_shape=jax.ShapeDtypeStruct((_NQ, _ND), jnp.float32),
    )(q_rank, d_rank_t)
    return out


# speedup vs baseline: 2.1326x; 2.1326x over previous
import jax
import jax.numpy as jnp
from jax.experimental import pallas as pl
from jax.experimental.pallas import tpu as pltpu
from jax.experimental.pallas import tpu_sc as plsc

_NA = 64
_ND = 4096
_NQ = 1024
_NR = _ND + _NQ
_RB = 512
_CB = 512
_LANES = 16

_CHUNK = 128
_SC_ROWS = 1024
_WPC = 4
_TPW = _NA // _WPC


def _logits_body(xt_ref, w_ref, b_ref, out_ref):
    lt = jax.lax.dot_general(
        w_ref[...], xt_ref[...], (((1,), (0,)), ((), ())),
        preferred_element_type=jnp.float32)
    out_ref[...] = lt + b_ref[...]


def _tree_count(srcs, va):
    terms = [jnp.where(s > va, 1.0, 0.0) for s in srcs]
    while len(terms) > 1:
        nxt = [terms[i] + terms[i + 1] for i in range(0, len(terms) - 1, 2)]
        if len(terms) % 2:
            nxt.append(terms[-1])
        terms = nxt
    return terms[0]


def _sc_rank_body(lt_hbm, out_hbm, lt_v, rk_v):
    c = jax.lax.axis_index("c")
    s = jax.lax.axis_index("s")
    wid = s * 2 + c
    chunk = wid // _WPC
    tbase = (wid % _WPC) * _TPW
    cbase = pl.multiple_of(chunk * _CHUNK, _CHUNK)
    pltpu.sync_copy(lt_hbm.at[:, pl.ds(cbase, _CHUNK)], lt_v)

    def group(g, carry):
        col = g * _LANES

        def half(h):
            srcs = [lt_v[h * 32 + j, pl.ds(col, _LANES)] for j in range(32)]

            def target(t, carry2):
                a = tbase + t
                va = lt_v[a, pl.ds(col, _LANES)]
                cnt = _tree_count(srcs, va)
                if h == 0:
                    rk_v[t, pl.ds(col, _LANES)] = cnt + 1.0
                else:
                    rk_v[t, pl.ds(col, _LANES)] = (
                        rk_v[t, pl.ds(col, _LANES)] + cnt)
                return carry2

            jax.lax.fori_loop(0, _TPW, target, 0)

        half(0)
        half(1)
        return carry

    jax.lax.fori_loop(0, _CHUNK // _LANES, group, 0)
    rbase = pl.multiple_of(tbase, _TPW)
    pltpu.sync_copy(rk_v, out_hbm.at[pl.ds(rbase, _TPW), pl.ds(cbase, _CHUNK)])


def _sc_ranks(lt):
    mesh = plsc.VectorSubcoreMesh(core_axis_name="c", subcore_axis_name="s")
    return pl.kernel(
        _sc_rank_body,
        out_type=jax.ShapeDtypeStruct((_NA, _SC_ROWS), jnp.float32),
        mesh=mesh,
        scratch_types=[
            pltpu.VMEM((_NA, _CHUNK), jnp.float32),
            pltpu.VMEM((_TPW, _CHUNK), jnp.float32),
        ],
    )(lt)


def _tc_rank_body(lt_ref, out_ref):
    lt = lt_ref[...]
    rank = jnp.ones(lt.shape, jnp.float32)
    for a in range(_NA):
        rank += (lt[a:a + 1, :] > lt).astype(jnp.float32)
    out_ref[...] = rank.astype(jnp.bfloat16)


def _mm_body(q_ref, d_ref, out_ref):
    out_ref[...] = jax.lax.dot_general(
        q_ref[...], d_ref[...], (((1,), (0,)), ((), ())),
        preferred_element_type=jnp.float32)


def kernel(data, query, W, b):
    with jax.enable_x64(False):
        return _kernel32(data, query, W, b)


def _kernel32(data, query, W, b):
    rows_t = jnp.concatenate([data, query], axis=0).T
    logits_t = pl.pallas_call(
        _logits_body,
        grid=(_NR // _RB,),
        in_specs=[
            pl.BlockSpec((128, _RB), lambda i: (0, i)),
            pl.BlockSpec((_NA, 128), lambda i: (0, 0)),
            pl.BlockSpec((_NA, 1), lambda i: (0, 0)),
        ],
        out_specs=pl.BlockSpec((_NA, _RB), lambda i: (0, i)),
        out_shape=jax.ShapeDtypeStruct((_NA, _NR), jnp.float32),
    )(rows_t, W, b.reshape(_NA, 1))
    ranks_sc = _sc_ranks(logits_t)
    ranks_tc = pl.pallas_call(
        _tc_rank_body,
        grid=((_NR - _SC_ROWS) // _RB,),
        in_specs=[
            pl.BlockSpec((_NA, _RB), lambda i: (0, i + _SC_ROWS // _RB)),
        ],
        out_specs=pl.BlockSpec((_NA, _RB), lambda i: (0, i)),
        out_shape=jax.ShapeDtypeStruct((_NA, _NR - _SC_ROWS), jnp.bfloat16),
    )(logits_t)
    ranks_t = jnp.concatenate(
        [ranks_sc.astype(jnp.bfloat16), ranks_tc], axis=1)
    d_rank_t = ranks_t[:, :_ND]
    q_rank = ranks_t[:, _ND:].T
    out = pl.pallas_call(
        _mm_body,
        grid=(_ND // _CB,),
        in_specs=[
            pl.BlockSpec((_NQ, _NA), lambda j: (0, 0)),
            pl.BlockSpec((_NA, _CB), lambda j: (0, j)),
        ],
        out_specs=pl.BlockSpec((_NQ, _CB), lambda j: (0, j)),
        out_shape=jax.ShapeDtypeStruct((_NQ, _ND), jnp.float32),
    )(q_rank, d_rank_t)
    return out


# speedup vs baseline: 2.7834x; 1.3052x over previous
import jax
import jax.numpy as jnp
from jax.experimental import pallas as pl
from jax.experimental.pallas import tpu as pltpu
from jax.experimental.pallas import tpu_sc as plsc

_NA = 64
_ND = 4096
_NQ = 1024
_RB = 512
_CB = 512
_LANES = 16

_CHUNK = 128
_SC_ROWS = 512
_WPC = 8
_TPW = _NA // _WPC
_TC_DROWS = _ND - _SC_ROWS


def _logits_body(x_ref, w_ref, b_ref, out_ref):
    lt = jax.lax.dot_general(
        w_ref[...], x_ref[...], (((1,), (1,)), ((), ())),
        preferred_element_type=jnp.float32)
    out_ref[...] = lt + b_ref[...]


def _tree_count(srcs, va):
    terms = [jnp.where(s > va, 1.0, 0.0) for s in srcs]
    while len(terms) > 1:
        nxt = [terms[i] + terms[i + 1] for i in range(0, len(terms) - 1, 2)]
        if len(terms) % 2:
            nxt.append(terms[-1])
        terms = nxt
    return terms[0]


def _sc_rank_body(lt_hbm, out_hbm, lt_v, rk_v):
    c = jax.lax.axis_index("c")
    s = jax.lax.axis_index("s")
    wid = s * 2 + c
    chunk = wid // _WPC
    tbase = (wid % _WPC) * _TPW
    cbase = pl.multiple_of(chunk * _CHUNK, _CHUNK)
    pltpu.sync_copy(lt_hbm.at[:, pl.ds(cbase, _CHUNK)], lt_v)

    def group(g, carry):
        col = g * _LANES

        def half(h):
            srcs = [lt_v[h * 32 + j, pl.ds(col, _LANES)] for j in range(32)]

            def target(t, carry2):
                a = tbase + t
                va = lt_v[a, pl.ds(col, _LANES)]
                cnt = _tree_count(srcs, va)
                if h == 0:
                    rk_v[t, pl.ds(col, _LANES)] = cnt + 1.0
                else:
                    rk_v[t, pl.ds(col, _LANES)] = (
                        rk_v[t, pl.ds(col, _LANES)] + cnt)
                return carry2

            jax.lax.fori_loop(0, _TPW, target, 0)

        half(0)
        half(1)
        return carry

    jax.lax.fori_loop(0, _CHUNK // _LANES, group, 0)
    rbase = pl.multiple_of(tbase, _TPW)
    pltpu.sync_copy(rk_v, out_hbm.at[pl.ds(rbase, _TPW), pl.ds(cbase, _CHUNK)])


def _sc_ranks(lt):
    mesh = plsc.VectorSubcoreMesh(core_axis_name="c", subcore_axis_name="s")
    return pl.kernel(
        _sc_rank_body,
        out_type=jax.ShapeDtypeStruct((_NA, _SC_ROWS), jnp.float32),
        mesh=mesh,
        scratch_types=[
            pltpu.VMEM((_NA, _CHUNK), jnp.float32),
            pltpu.VMEM((_TPW, _CHUNK), jnp.float32),
        ],
    )(lt)


def _rank_fused_body(x_ref, w_ref, b_ref, out_ref):
    lt = jax.lax.dot_general(
        w_ref[...], x_ref[...], (((1,), (1,)), ((), ())),
        preferred_element_type=jnp.float32)
    lt = lt + b_ref[...]
    rank = jnp.ones(lt.shape, jnp.float32)
    for a in range(_NA):
        rank += (lt[a:a + 1, :] > lt).astype(jnp.float32)
    out_ref[...] = rank.astype(jnp.bfloat16)


def _mm_body(q_ref, dsc_ref, dtc_ref, out_ref):
    j = pl.program_id(0)
    d = jnp.where(j == 0, dsc_ref[...].astype(jnp.bfloat16), dtc_ref[...])
    out_ref[...] = jax.lax.dot_general(
        q_ref[...], d, (((0,), (0,)), ((), ())),
        preferred_element_type=jnp.float32)


def kernel(data, query, W, b):
    with jax.enable_x64(False):
        return _kernel32(data, query, W, b)


def _kernel32(data, query, W, b):
    b2 = b.reshape(_NA, 1)
    lt_sc = pl.pallas_call(
        _logits_body,
        grid=(1,),
        in_specs=[
            pl.BlockSpec((_SC_ROWS, 128), lambda i: (0, 0)),
            pl.BlockSpec((_NA, 128), lambda i: (0, 0)),
            pl.BlockSpec((_NA, 1), lambda i: (0, 0)),
        ],
        out_specs=pl.BlockSpec((_NA, _SC_ROWS), lambda i: (0, 0)),
        out_shape=jax.ShapeDtypeStruct((_NA, _SC_ROWS), jnp.float32),
    )(data, W, b2)
    ranks_sc = _sc_ranks(lt_sc)
    ranks_dtc = pl.pallas_call(
        _rank_fused_body,
        grid=(_TC_DROWS // _RB,),
        in_specs=[
            pl.BlockSpec((_RB, 128), lambda i: (i + 1, 0)),
            pl.BlockSpec((_NA, 128), lambda i: (0, 0)),
            pl.BlockSpec((_NA, 1), lambda i: (0, 0)),
        ],
        out_specs=pl.BlockSpec((_NA, _RB), lambda i: (0, i)),
        out_shape=jax.ShapeDtypeStruct((_NA, _TC_DROWS), jnp.bfloat16),
    )(data, W, b2)
    ranks_q = pl.pallas_call(
        _rank_fused_body,
        grid=(_NQ // _RB,),
        in_specs=[
            pl.BlockSpec((_RB, 128), lambda i: (i, 0)),
            pl.BlockSpec((_NA, 128), lambda i: (0, 0)),
            pl.BlockSpec((_NA, 1), lambda i: (0, 0)),
        ],
        out_specs=pl.BlockSpec((_NA, _RB), lambda i: (0, i)),
        out_shape=jax.ShapeDtypeStruct((_NA, _NQ), jnp.bfloat16),
    )(query, W, b2)
    out = pl.pallas_call(
        _mm_body,
        grid=(_ND // _CB,),
        in_specs=[
            pl.BlockSpec((_NA, _NQ), lambda j: (0, 0)),
            pl.BlockSpec((_NA, _SC_ROWS), lambda j: (0, 0)),
            pl.BlockSpec((_NA, _CB), lambda j: (0, jnp.maximum(j - 1, 0))),
        ],
        out_specs=pl.BlockSpec((_NQ, _CB), lambda j: (0, j)),
        out_shape=jax.ShapeDtypeStruct((_NQ, _ND), jnp.float32),
    )(ranks_q, ranks_sc, ranks_dtc)
    return out


# speedup vs baseline: 2.8839x; 1.0361x over previous
import jax
import jax.numpy as jnp
from jax.experimental import pallas as pl
from jax.experimental.pallas import tpu as pltpu
from jax.experimental.pallas import tpu_sc as plsc

_NA = 64
_ND = 4096
_NQ = 1024
_RB = 512
_CB = 512
_LANES = 16

_CHUNK = 128
_SC_ROWS = 512
_WPC = 8
_TPW = _NA // _WPC
_TC_DROWS = _ND - _SC_ROWS


def _logits_body(x_ref, w_ref, b_ref, out_ref):
    lt = jax.lax.dot_general(
        w_ref[...], x_ref[...], (((1,), (1,)), ((), ())),
        preferred_element_type=jnp.float32)
    out_ref[...] = lt + b_ref[...]


def _tree_count(srcs, va):
    terms = [jnp.where(s > va, 1.0, 0.0) for s in srcs]
    while len(terms) > 1:
        nxt = [terms[i] + terms[i + 1] for i in range(0, len(terms) - 1, 2)]
        if len(terms) % 2:
            nxt.append(terms[-1])
        terms = nxt
    return terms[0]


def _sc_rank_body(lt_hbm, out_hbm, lt_v, rk_v):
    c = jax.lax.axis_index("c")
    s = jax.lax.axis_index("s")
    wid = s * 2 + c
    chunk = wid // _WPC
    tbase = (wid % _WPC) * _TPW
    cbase = pl.multiple_of(chunk * _CHUNK, _CHUNK)
    pltpu.sync_copy(lt_hbm.at[:, pl.ds(cbase, _CHUNK)], lt_v)

    def group(g, carry):
        col = g * _LANES

        def half(h):
            srcs = [lt_v[h * 32 + j, pl.ds(col, _LANES)] for j in range(32)]

            def target(t, carry2):
                a = tbase + t
                va = lt_v[a, pl.ds(col, _LANES)]
                cnt = _tree_count(srcs, va)
                if h == 0:
                    rk_v[t, pl.ds(col, _LANES)] = cnt + 1.0
                else:
                    rk_v[t, pl.ds(col, _LANES)] = (
                        rk_v[t, pl.ds(col, _LANES)] + cnt)
                return carry2

            jax.lax.fori_loop(0, _TPW, target, 0)

        half(0)
        half(1)
        return carry

    jax.lax.fori_loop(0, _CHUNK // _LANES, group, 0)
    rbase = pl.multiple_of(tbase, _TPW)
    pltpu.sync_copy(rk_v, out_hbm.at[pl.ds(rbase, _TPW), pl.ds(cbase, _CHUNK)])


def _sc_ranks(lt):
    mesh = plsc.VectorSubcoreMesh(core_axis_name="c", subcore_axis_name="s")
    return pl.kernel(
        _sc_rank_body,
        out_type=jax.ShapeDtypeStruct((_NA, _SC_ROWS), jnp.float32),
        mesh=mesh,
        scratch_types=[
            pltpu.VMEM((_NA, _CHUNK), jnp.float32),
            pltpu.VMEM((_TPW, _CHUNK), jnp.float32),
        ],
    )(lt)


def _rank_fused_body(q_ref, d_ref, w_ref, b_ref, out_ref):
    i = pl.program_id(0)
    x = jnp.where(i < _NQ // _RB, q_ref[...], d_ref[...])
    lt = jax.lax.dot_general(
        w_ref[...], x, (((1,), (1,)), ((), ())),
        preferred_element_type=jnp.float32)
    lt = lt + b_ref[...]
    rank = jnp.ones(lt.shape, jnp.float32)
    for a in range(_NA):
        rank += (lt[a:a + 1, :] > lt).astype(jnp.float32)
    out_ref[...] = rank.astype(jnp.bfloat16)


def _mm_body(q_ref, dsc_ref, dtc_ref, out_ref):
    j = pl.program_id(0)
    d = jnp.where(j == 0, dsc_ref[...].astype(jnp.bfloat16), dtc_ref[...])
    out_ref[...] = jax.lax.dot_general(
        q_ref[...], d, (((0,), (0,)), ((), ())),
        preferred_element_type=jnp.float32)


def kernel(data, query, W, b):
    with jax.enable_x64(False):
        return _kernel32(data, query, W, b)


def _kernel32(data, query, W, b):
    b2 = b.reshape(_NA, 1)
    lt_sc = pl.pallas_call(
        _logits_body,
        grid=(1,),
        in_specs=[
            pl.BlockSpec((_SC_ROWS, 128), lambda i: (0, 0)),
            pl.BlockSpec((_NA, 128), lambda i: (0, 0)),
            pl.BlockSpec((_NA, 1), lambda i: (0, 0)),
        ],
        out_specs=pl.BlockSpec((_NA, _SC_ROWS), lambda i: (0, 0)),
        out_shape=jax.ShapeDtypeStruct((_NA, _SC_ROWS), jnp.float32),
    )(data, W, b2)
    ranks_sc = _sc_ranks(lt_sc)
    ranks_tc = pl.pallas_call(
        _rank_fused_body,
        grid=((_NQ + _TC_DROWS) // _RB,),
        in_specs=[
            pl.BlockSpec((_RB, 128), lambda i: (jnp.minimum(i, 1), 0)),
            pl.BlockSpec(
                (_RB, 128), lambda i: (jnp.clip(i - 1, 1, _ND // _RB - 1), 0)),
            pl.BlockSpec((_NA, 128), lambda i: (0, 0)),
            pl.BlockSpec((_NA, 1), lambda i: (0, 0)),
        ],
        out_specs=pl.BlockSpec((_NA, _RB), lambda i: (0, i)),
        out_shape=jax.ShapeDtypeStruct(
            (_NA, _NQ + _TC_DROWS), jnp.bfloat16),
    )(query, data, W, b2)
    out = pl.pallas_call(
        _mm_body,
        grid=(_ND // _CB,),
        in_specs=[
            pl.BlockSpec((_NA, _NQ), lambda j: (0, 0)),
            pl.BlockSpec((_NA, _SC_ROWS), lambda j: (0, 0)),
            pl.BlockSpec(
                (_NA, _CB),
                lambda j: (0, jnp.maximum(j + 1, 2))),
        ],
        out_specs=pl.BlockSpec((_NQ, _CB), lambda j: (0, j)),
        out_shape=jax.ShapeDtypeStruct((_NQ, _ND), jnp.float32),
    )(ranks_tc, ranks_sc, ranks_tc)
    return out


# speedup vs baseline: 2.8902x; 1.0022x over previous
import jax
import jax.numpy as jnp
from jax.experimental import pallas as pl
from jax.experimental.pallas import tpu as pltpu
from jax.experimental.pallas import tpu_sc as plsc

_NA = 64
_ND = 4096
_NQ = 1024
_RB = 512
_CB = 512
_LANES = 16

_CHUNK = 128
_SC_ROWS = 512
_WPC = 8
_TPW = _NA // _WPC
_TC_DROWS = _ND - _SC_ROWS


def _logits_body(x_ref, w_ref, b_ref, out_ref):
    lt = jax.lax.dot_general(
        w_ref[...], x_ref[...], (((1,), (1,)), ((), ())),
        preferred_element_type=jnp.float32)
    out_ref[...] = lt + b_ref[...]


def _tree_count(srcs, va):
    terms = [jnp.where(s > va, 1.0, 0.0) for s in srcs]
    while len(terms) > 1:
        nxt = [terms[i] + terms[i + 1] for i in range(0, len(terms) - 1, 2)]
        if len(terms) % 2:
            nxt.append(terms[-1])
        terms = nxt
    return terms[0]


def _sc_rank_body(lt_hbm, out_hbm, lt_v, rk_v):
    c = jax.lax.axis_index("c")
    s = jax.lax.axis_index("s")
    wid = s * 2 + c
    chunk = wid // _WPC
    tbase = (wid % _WPC) * _TPW
    cbase = pl.multiple_of(chunk * _CHUNK, _CHUNK)
    pltpu.sync_copy(lt_hbm.at[:, pl.ds(cbase, _CHUNK)], lt_v)

    def group(g, carry):
        col = g * _LANES

        def target(t, carry2):
            a = tbase + t
            va = lt_v[a, pl.ds(col, _LANES)]
            srcs = [lt_v[j, pl.ds(col, _LANES)] for j in range(_NA)]
            rk_v[t, pl.ds(col, _LANES)] = _tree_count(srcs, va) + 1.0
            return carry2

        jax.lax.fori_loop(0, _TPW, target, 0)
        return carry

    jax.lax.fori_loop(0, _CHUNK // _LANES, group, 0)
    rbase = pl.multiple_of(tbase, _TPW)
    pltpu.sync_copy(rk_v, out_hbm.at[pl.ds(rbase, _TPW), pl.ds(cbase, _CHUNK)])


def _sc_ranks(lt):
    mesh = plsc.VectorSubcoreMesh(core_axis_name="c", subcore_axis_name="s")
    return pl.kernel(
        _sc_rank_body,
        out_type=jax.ShapeDtypeStruct((_NA, _SC_ROWS), jnp.float32),
        mesh=mesh,
        scratch_types=[
            pltpu.VMEM((_NA, _CHUNK), jnp.float32),
            pltpu.VMEM((_TPW, _CHUNK), jnp.float32),
        ],
    )(lt)


def _rank_fused_body(q_ref, d_ref, w_ref, b_ref, out_ref):
    i = pl.program_id(0)
    x = jnp.where(i < _NQ // _RB, q_ref[...], d_ref[...])
    lt = jax.lax.dot_general(
        w_ref[...], x, (((1,), (1,)), ((), ())),
        preferred_element_type=jnp.float32)
    lt = lt + b_ref[...]
    rank = jnp.ones(lt.shape, jnp.float32)
    for a in range(_NA):
        rank += (lt[a:a + 1, :] > lt).astype(jnp.float32)
    out_ref[...] = rank.astype(jnp.bfloat16)


def _mm_body(q_ref, dsc_ref, dtc_ref, out_ref):
    j = pl.program_id(0)
    d = jnp.where(j == 0, dsc_ref[...].astype(jnp.bfloat16), dtc_ref[...])
    out_ref[...] = jax.lax.dot_general(
        q_ref[...], d, (((0,), (0,)), ((), ())),
        preferred_element_type=jnp.float32)


def kernel(data, query, W, b):
    with jax.enable_x64(False):
        return _kernel32(data, query, W, b)


def _kernel32(data, query, W, b):
    b2 = b.reshape(_NA, 1)
    lt_sc = pl.pallas_call(
        _logits_body,
        grid=(1,),
        in_specs=[
            pl.BlockSpec((_SC_ROWS, 128), lambda i: (0, 0)),
            pl.BlockSpec((_NA, 128), lambda i: (0, 0)),
            pl.BlockSpec((_NA, 1), lambda i: (0, 0)),
        ],
        out_specs=pl.BlockSpec((_NA, _SC_ROWS), lambda i: (0, 0)),
        out_shape=jax.ShapeDtypeStruct((_NA, _SC_ROWS), jnp.float32),
    )(data, W, b2)
    ranks_sc = _sc_ranks(lt_sc)
    ranks_tc = pl.pallas_call(
        _rank_fused_body,
        grid=((_NQ + _TC_DROWS) // _RB,),
        in_specs=[
            pl.BlockSpec((_RB, 128), lambda i: (jnp.minimum(i, 1), 0)),
            pl.BlockSpec(
                (_RB, 128), lambda i: (jnp.clip(i - 1, 1, _ND // _RB - 1), 0)),
            pl.BlockSpec((_NA, 128), lambda i: (0, 0)),
            pl.BlockSpec((_NA, 1), lambda i: (0, 0)),
        ],
        out_specs=pl.BlockSpec((_NA, _RB), lambda i: (0, i)),
        out_shape=jax.ShapeDtypeStruct(
            (_NA, _NQ + _TC_DROWS), jnp.bfloat16),
    )(query, data, W, b2)
    out = pl.pallas_call(
        _mm_body,
        grid=(_ND // _CB,),
        in_specs=[
            pl.BlockSpec((_NA, _NQ), lambda j: (0, 0)),
            pl.BlockSpec((_NA, _SC_ROWS), lambda j: (0, 0)),
            pl.BlockSpec(
                (_NA, _CB),
                lambda j: (0, jnp.maximum(j + 1, 2))),
        ],
        out_specs=pl.BlockSpec((_NQ, _CB), lambda j: (0, j)),
        out_shape=jax.ShapeDtypeStruct((_NQ, _ND), jnp.float32),
    )(ranks_tc, ranks_sc, ranks_tc)
    return out
